# baseline (speedup 1.0000x reference)
import functools

import jax
import jax.numpy as jnp
from jax import lax
from jax.experimental import pallas as pl
from jax.experimental.pallas import tpu as pltpu
from jax.experimental.pallas import tpu_sc as plsc

N = 10000
E = 320000
F = 128
H = 256
O = 2

NC = 2
NS = 16
NW = NC * NS
L = 16
NP = 10240
RPT = NP // NS
EPW = E // NW
SLAB = 512
NSLAB = 20
E3 = NW * NSLAB * SLAB
NPF = 2 * NP


def _mesh():
    return plsc.VectorSubcoreMesh(core_axis_name="c", subcore_axis_name="s")


def _sc_degrees(src, dst, zflat):
    @functools.partial(
        pl.kernel,
        mesh=_mesh(),
        compiler_params=pltpu.CompilerParams(needs_layout_passes=False),
        out_type=jax.ShapeDtypeStruct((NW, NPF), jnp.float32),
        scratch_types=[
            pltpu.VMEM((EPW,), jnp.int32),
            pltpu.VMEM((EPW,), jnp.int32),
            pltpu.VMEM((NPF,), jnp.float32),
        ],
    )
    def k(src_hbm, dst_hbm, zf_hbm, out_hbm, sidx_v, didx_v, cnt_v):
        c = lax.axis_index("c")
        s = lax.axis_index("s")
        wid = s * NC + c
        base = wid * EPW
        pltpu.sync_copy(zf_hbm, cnt_v)
        pltpu.sync_copy(src_hbm.at[pl.ds(base, EPW)], sidx_v)
        pltpu.sync_copy(dst_hbm.at[pl.ds(base, EPW)], didx_v)
        ones = jnp.ones((L,), jnp.float32)
        offs = jnp.full((L,), NP, jnp.int32)

        def body(g, carry):
            i0 = g * L
            plsc.addupdate_scatter(cnt_v, [sidx_v[pl.ds(i0, L)]], ones)
            plsc.addupdate_scatter(cnt_v, [didx_v[pl.ds(i0, L)] + offs], ones)
            return carry

        lax.fori_loop(0, EPW // L, body, 0)
        pltpu.sync_copy(cnt_v, out_hbm.at[wid])

    return k(src, dst, zflat)


def _sc_prop_half(xs, srcp, dstp, z2d):
    NBUF = 2
    FH = F // 2

    @functools.partial(
        pl.kernel,
        mesh=_mesh(),
        compiler_params=pltpu.CompilerParams(
            needs_layout_passes=False, use_tc_tiling_on_sc=False
        ),
        out_type=jax.ShapeDtypeStruct((NC, NP, FH), jnp.float32),
        scratch_types=[
            pltpu.VMEM((NSLAB + 1, SLAB), jnp.int32),
            pltpu.VMEM((NSLAB, SLAB), jnp.int32),
            pltpu.VMEM((NBUF, SLAB, FH), jnp.float32),
            pltpu.VMEM_SHARED((NP, FH), jnp.float32),
        ]
        + [pltpu.SemaphoreType.DMA] * (2 * NBUF),
    )
    def k(xs_hbm, sp_hbm, dp_hbm, z_hbm, out_hbm, src_v, dst_v, rows_v, acc_sh,
          g0, g1, s0, s1):
        gsem = (g0, g1)
        ssem = (s0, s1)
        c = lax.axis_index("c")
        s = lax.axis_index("s")
        wid = s * NC + c
        pltpu.sync_copy(z_hbm, acc_sh.at[pl.ds(s * RPT, RPT)])
        pltpu.sync_copy(sp_hbm.at[wid], src_v)
        pltpu.sync_copy(dp_hbm.at[wid], dst_v)
        plsc.subcore_barrier()

        def start_gather(j, t):
            pltpu.async_copy(
                xs_hbm.at[src_v.at[j]], rows_v.at[t], gsem[t]
            )

        def wait_gather(j, t):
            pltpu.make_async_copy(
                xs_hbm.at[src_v.at[j]], rows_v.at[t], gsem[t]
            ).wait()

        def start_scatter(j, t):
            pltpu.async_copy(
                rows_v.at[t], acc_sh.at[dst_v.at[j]], ssem[t],
                add=True,
            )

        def wait_scatter(j, t):
            pltpu.make_async_copy(
                rows_v.at[t], acc_sh.at[dst_v.at[j]], ssem[t]
            ).wait()

        start_gather(0, 0)
        start_gather(1, 1)
        wait_gather(0, 0)
        start_scatter(0, 0)

        def body(m, carry):
            j1 = 2 * m + 1
            wait_scatter(j1 - 1, 0)
            start_gather(j1 + 1, 0)
            wait_gather(j1, 1)
            start_scatter(j1, 1)
            j2 = j1 + 1
            wait_scatter(j2 - 1, 1)
            start_gather(j2 + 1, 1)
            wait_gather(j2, 0)
            start_scatter(j2, 0)
            return carry

        lax.fori_loop(0, (NSLAB - 2) // 2, body, 0)
        j = NSLAB - 1
        wait_scatter(j - 1, 0)
        start_gather(j + 1, 0)
        wait_gather(j, 1)
        start_scatter(j, 1)
        wait_scatter(j, 1)
        wait_gather(NSLAB, 0)
        plsc.subcore_barrier()
        pltpu.sync_copy(
            acc_sh.at[pl.ds(s * RPT, RPT)], out_hbm.at[c, pl.ds(s * RPT, RPT)]
        )

    return k(xs, srcp, dstp, z2d)


def _sc_prop2(gflat, src, dst, zflat):
    @functools.partial(
        pl.kernel,
        mesh=_mesh(),
        compiler_params=pltpu.CompilerParams(needs_layout_passes=False),
        out_type=jax.ShapeDtypeStruct((NW, NPF), jnp.float32),
        scratch_types=[
            pltpu.VMEM((NPF,), jnp.float32),
            pltpu.VMEM((NPF,), jnp.float32),
            pltpu.VMEM((EPW,), jnp.int32),
            pltpu.VMEM((EPW,), jnp.int32),
        ],
    )
    def k(g_hbm, src_hbm, dst_hbm, zf_hbm, out_hbm, g_v, acc_v, sidx_v, didx_v):
        c = lax.axis_index("c")
        s = lax.axis_index("s")
        wid = s * NC + c
        base = wid * EPW
        pltpu.sync_copy(zf_hbm, acc_v)
        pltpu.sync_copy(g_hbm, g_v)
        pltpu.sync_copy(src_hbm.at[pl.ds(base, EPW)], sidx_v)
        pltpu.sync_copy(dst_hbm.at[pl.ds(base, EPW)], didx_v)
        ones = jnp.full((L,), 1, jnp.int32)

        def body(g, carry):
            i0 = g * L
            si = sidx_v[pl.ds(i0, L)] * 2
            di = didx_v[pl.ds(i0, L)] * 2
            v0 = plsc.load_gather(g_v, [si])
            v1 = plsc.load_gather(g_v, [si + ones])
            plsc.addupdate_scatter(acc_v, [di], v0)
            plsc.addupdate_scatter(acc_v, [di + ones], v1)
            return carry

        lax.fori_loop(0, EPW // L, body, 0)
        pltpu.sync_copy(acc_v, out_hbm.at[wid])

    return k(gflat, src, dst, zflat)


def _tc_count_norm(cnt32):
    def body(c_ref, o_ref):
        o_ref[...] = lax.rsqrt(jnp.maximum(jnp.sum(c_ref[...], axis=0), 1.0))

    return pl.pallas_call(
        body,
        out_shape=jax.ShapeDtypeStruct((NPF // 128, 128), jnp.float32),
    )(cnt32)


def _tc_scale(x_pad, nsrc_col):
    FH = F // 2

    def body(x_ref, n_ref, o0_ref, o1_ref):
        xs = x_ref[...] * n_ref[...]
        o0_ref[...] = xs[:, :FH]
        o1_ref[...] = xs[:, FH:]

    return pl.pallas_call(
        body,
        out_shape=(
            jax.ShapeDtypeStruct((NP, FH), jnp.float32),
            jax.ShapeDtypeStruct((NP, FH), jnp.float32),
        ),
    )(x_pad, nsrc_col)


def _tc_mlp(P0, P1, ndst_col, nsrc_col, W1, b1r, W2):
    R = 1024
    NB = NP // R
    FH = F // 2

    def body(p0_ref, p1_ref, nd_ref, ns_ref, w1_ref, b1_ref, w2_ref, o_ref):
        nd = nd_ref[...]
        m0 = (p0_ref[0] + p0_ref[1]) * nd
        m1 = (p1_ref[0] + p1_ref[1]) * nd
        w1 = w1_ref[...]
        h = jnp.dot(m0, w1[:FH, :], preferred_element_type=jnp.float32)
        h = h + jnp.dot(m1, w1[FH:, :], preferred_element_type=jnp.float32)
        h = jnp.maximum(h + b1_ref[...], 0.0)
        o_ref[...] = jnp.dot(
            h * ns_ref[...], w2_ref[...], preferred_element_type=jnp.float32
        )

    return pl.pallas_call(
        body,
        grid=(NB,),
        in_specs=[
            pl.BlockSpec((NC, R, FH), lambda i: (0, i, 0)),
            pl.BlockSpec((NC, R, FH), lambda i: (0, i, 0)),
            pl.BlockSpec((R, 1), lambda i: (i, 0)),
            pl.BlockSpec((R, 1), lambda i: (i, 0)),
            pl.BlockSpec((F, H), lambda i: (0, 0)),
            pl.BlockSpec((1, H), lambda i: (0, 0)),
            pl.BlockSpec((H, O), lambda i: (0, 0)),
        ],
        out_specs=pl.BlockSpec((R, O), lambda i: (i, 0)),
        out_shape=jax.ShapeDtypeStruct((NP, O), jnp.float32),
    )(P0, P1, ndst_col, nsrc_col, W1, b1r, W2)


def _tc_sum(q32):
    def body(q_ref, o_ref):
        o_ref[...] = jnp.sum(q_ref[...], axis=0)

    return pl.pallas_call(
        body,
        out_shape=jax.ShapeDtypeStruct((NPF // 128, 128), jnp.float32),
    )(q32)


def _tc_final(q2, ndst_col, b2r):
    def body(q_ref, nd_ref, b_ref, o_ref):
        o_ref[...] = q_ref[...] * nd_ref[...] + b_ref[...]

    return pl.pallas_call(
        body,
        out_shape=jax.ShapeDtypeStruct((NP, O), jnp.float32),
    )(q2, ndst_col, b2r)


def kernel(inputs, edge_index, W1, b1, W2, b2):
    src = edge_index[0].astype(jnp.int32)
    dst = edge_index[1].astype(jnp.int32)
    zflat = jnp.zeros((NPF,), jnp.float32)
    z2d = jnp.zeros((RPT, F // 2), jnp.float32)
    x_pad = jnp.pad(inputs, ((0, NP - N), (0, 0)))

    cnt32 = _sc_degrees(src, dst, zflat)
    norms2d = _tc_count_norm(cnt32.reshape(NW, NPF // 128, 128))
    norms = norms2d.reshape(NPF, 1)
    nsrc_col = norms[:NP]
    ndst_col = norms[NP:]
    xs0, xs1 = _tc_scale(x_pad, nsrc_col)

    src2 = jnp.concatenate([src, jnp.zeros((E3 - E,), jnp.int32)]).reshape(
        NW, NSLAB, SLAB
    )
    srcp = jnp.concatenate([src2, jnp.zeros((NW, 1, SLAB), jnp.int32)], axis=1)
    dstp = jnp.concatenate([dst, jnp.full((E3 - E,), N, jnp.int32)]).reshape(
        NW, NSLAB, SLAB
    )
    P0 = _sc_prop_half(xs0, srcp, dstp, z2d)
    P1 = _sc_prop_half(xs1, srcp, dstp, z2d)

    g = _tc_mlp(P0, P1, ndst_col, nsrc_col, W1, b1.reshape(1, H), W2)
    q32 = _sc_prop2(g.reshape(NPF), src, dst, zflat)
    q2d = _tc_sum(q32.reshape(NW, NPF // 128, 128))
    out = _tc_final(q2d.reshape(NP, O), ndst_col, b2.reshape(1, O))
    return out[:N]


# speedup vs baseline: 3.8263x; 3.8263x over previous
import functools

import jax
import jax.numpy as jnp
from jax import lax
from jax.experimental import pallas as pl
from jax.experimental.pallas import tpu as pltpu
from jax.experimental.pallas import tpu_sc as plsc

N = 10000
E = 320000
F = 128
H = 256
O = 2

NC = 2
NS = 16
NW = NC * NS
L = 16
NP = 10240
RPT = NP // NS
EPW = E // NW
C3 = 128
K3 = 80
E3 = NW * K3 * C3
NPF = 2 * NP


def _mesh():
    return plsc.VectorSubcoreMesh(core_axis_name="c", subcore_axis_name="s")


def _sc_degrees(src, dst, zflat):
    @functools.partial(
        pl.kernel,
        mesh=_mesh(),
        compiler_params=pltpu.CompilerParams(needs_layout_passes=False),
        out_type=jax.ShapeDtypeStruct((NW, NPF), jnp.float32),
        scratch_types=[
            pltpu.VMEM((EPW,), jnp.int32),
            pltpu.VMEM((EPW,), jnp.int32),
            pltpu.VMEM((NPF,), jnp.float32),
        ],
    )
    def k(src_hbm, dst_hbm, zf_hbm, out_hbm, sidx_v, didx_v, cnt_v):
        c = lax.axis_index("c")
        s = lax.axis_index("s")
        wid = s * NC + c
        base = wid * EPW
        pltpu.sync_copy(zf_hbm, cnt_v)
        pltpu.sync_copy(src_hbm.at[pl.ds(base, EPW)], sidx_v)
        pltpu.sync_copy(dst_hbm.at[pl.ds(base, EPW)], didx_v)
        ones = jnp.ones((L,), jnp.float32)
        offs = jnp.full((L,), NP, jnp.int32)

        def body(g, carry):
            i0 = g * L
            plsc.addupdate_scatter(cnt_v, [sidx_v[pl.ds(i0, L)]], ones)
            plsc.addupdate_scatter(cnt_v, [didx_v[pl.ds(i0, L)] + offs], ones)
            return carry

        lax.fori_loop(0, EPW // L, body, 0)
        pltpu.sync_copy(cnt_v, out_hbm.at[wid])

    return k(src, dst, zflat)


def _sc_prop128(xs, srcp, dstp, z2d):
    @functools.partial(
        pl.kernel,
        mesh=_mesh(),
        compiler_params=pltpu.CompilerParams(needs_layout_passes=False),
        out_type=jax.ShapeDtypeStruct((NC, NP, F), jnp.float32),
        scratch_types=[
            pltpu.VMEM((K3, C3), jnp.int32),
            pltpu.VMEM((K3, C3), jnp.int32),
            pltpu.VMEM((C3, F), jnp.float32),
            pltpu.VMEM_SHARED((NP, F), jnp.float32),
            pltpu.SemaphoreType.DMA,
        ],
    )
    def k(xs_hbm, sp_hbm, dp_hbm, z_hbm, out_hbm, src_v, dst_v, rows_v, acc_sh, sem):
        c = lax.axis_index("c")
        s = lax.axis_index("s")
        wid = s * NC + c
        pltpu.sync_copy(z_hbm, acc_sh.at[pl.ds(s * RPT, RPT)])
        pltpu.sync_copy(sp_hbm.at[wid], src_v)
        pltpu.sync_copy(dp_hbm.at[wid], dst_v)
        plsc.subcore_barrier()

        def body(j, carry):
            pltpu.async_copy(xs_hbm.at[src_v.at[j]], rows_v, sem).wait()
            pltpu.sync_copy(rows_v, acc_sh.at[dst_v.at[j]], add=True)
            return carry

        lax.fori_loop(0, K3, body, 0)
        plsc.subcore_barrier()
        pltpu.sync_copy(
            acc_sh.at[pl.ds(s * RPT, RPT)], out_hbm.at[c, pl.ds(s * RPT, RPT)]
        )

    return k(xs, srcp, dstp, z2d)


def _sc_prop2(gflat, src, dst, zflat):
    @functools.partial(
        pl.kernel,
        mesh=_mesh(),
        compiler_params=pltpu.CompilerParams(needs_layout_passes=False),
        out_type=jax.ShapeDtypeStruct((NW, NPF), jnp.float32),
        scratch_types=[
            pltpu.VMEM((NPF,), jnp.float32),
            pltpu.VMEM((NPF,), jnp.float32),
            pltpu.VMEM((EPW,), jnp.int32),
            pltpu.VMEM((EPW,), jnp.int32),
        ],
    )
    def k(g_hbm, src_hbm, dst_hbm, zf_hbm, out_hbm, g_v, acc_v, sidx_v, didx_v):
        c = lax.axis_index("c")
        s = lax.axis_index("s")
        wid = s * NC + c
        base = wid * EPW
        pltpu.sync_copy(zf_hbm, acc_v)
        pltpu.sync_copy(g_hbm, g_v)
        pltpu.sync_copy(src_hbm.at[pl.ds(base, EPW)], sidx_v)
        pltpu.sync_copy(dst_hbm.at[pl.ds(base, EPW)], didx_v)
        ones = jnp.full((L,), 1, jnp.int32)

        def body(g, carry):
            i0 = g * L
            si = sidx_v[pl.ds(i0, L)] * 2
            di = didx_v[pl.ds(i0, L)] * 2
            v0 = plsc.load_gather(g_v, [si])
            v1 = plsc.load_gather(g_v, [si + ones])
            plsc.addupdate_scatter(acc_v, [di], v0)
            plsc.addupdate_scatter(acc_v, [di + ones], v1)
            return carry

        lax.fori_loop(0, EPW // L, body, 0)
        pltpu.sync_copy(acc_v, out_hbm.at[wid])

    return k(gflat, src, dst, zflat)


def _tc_count_norm(cnt32):
    def body(c_ref, o_ref):
        o_ref[...] = lax.rsqrt(jnp.maximum(jnp.sum(c_ref[...], axis=0), 1.0))

    return pl.pallas_call(
        body,
        out_shape=jax.ShapeDtypeStruct((NPF // 128, 128), jnp.float32),
    )(cnt32)


def _tc_scale(x_pad, nsrc_col):
    def body(x_ref, n_ref, o_ref):
        o_ref[...] = x_ref[...] * n_ref[...]

    return pl.pallas_call(
        body,
        out_shape=jax.ShapeDtypeStruct((NP, F), jnp.float32),
    )(x_pad, nsrc_col)


def _tc_mlp(P, ndst_col, nsrc_col, W1, b1r, W2):
    R = 1024
    NB = NP // R

    def body(p_ref, nd_ref, ns_ref, w1_ref, b1_ref, w2_ref, o_ref):
        m = (p_ref[0] + p_ref[1]) * nd_ref[...]
        h = jnp.dot(m, w1_ref[...], preferred_element_type=jnp.float32)
        h = jnp.maximum(h + b1_ref[...], 0.0)
        o_ref[...] = jnp.dot(
            h * ns_ref[...], w2_ref[...], preferred_element_type=jnp.float32
        )

    return pl.pallas_call(
        body,
        grid=(NB,),
        in_specs=[
            pl.BlockSpec((NC, R, F), lambda i: (0, i, 0)),
            pl.BlockSpec((R, 1), lambda i: (i, 0)),
            pl.BlockSpec((R, 1), lambda i: (i, 0)),
            pl.BlockSpec((F, H), lambda i: (0, 0)),
            pl.BlockSpec((1, H), lambda i: (0, 0)),
            pl.BlockSpec((H, O), lambda i: (0, 0)),
        ],
        out_specs=pl.BlockSpec((R, O), lambda i: (i, 0)),
        out_shape=jax.ShapeDtypeStruct((NP, O), jnp.float32),
    )(P, ndst_col, nsrc_col, W1, b1r, W2)


def _tc_sum(q32):
    def body(q_ref, o_ref):
        o_ref[...] = jnp.sum(q_ref[...], axis=0)

    return pl.pallas_call(
        body,
        out_shape=jax.ShapeDtypeStruct((NPF // 128, 128), jnp.float32),
    )(q32)


def _tc_final(q2, ndst_col, b2r):
    def body(q_ref, nd_ref, b_ref, o_ref):
        o_ref[...] = q_ref[...] * nd_ref[...] + b_ref[...]

    return pl.pallas_call(
        body,
        out_shape=jax.ShapeDtypeStruct((NP, O), jnp.float32),
    )(q2, ndst_col, b2r)


def kernel(inputs, edge_index, W1, b1, W2, b2):
    src = edge_index[0].astype(jnp.int32)
    dst = edge_index[1].astype(jnp.int32)
    zflat = jnp.zeros((NPF,), jnp.float32)
    z2d = jnp.zeros((RPT, F), jnp.float32)
    x_pad = jnp.pad(inputs, ((0, NP - N), (0, 0)))

    cnt32 = _sc_degrees(src, dst, zflat)
    norms2d = _tc_count_norm(cnt32.reshape(NW, NPF // 128, 128))
    norms = norms2d.reshape(NPF, 1)
    nsrc_col = norms[:NP]
    ndst_col = norms[NP:]
    xs = _tc_scale(x_pad, nsrc_col)

    npad = E3 - E
    pad_i = jnp.arange(npad, dtype=jnp.int32)
    srcp = jnp.concatenate([src, pad_i % N]).reshape(NW, K3, C3)
    dstp = jnp.concatenate([dst, N + pad_i % (NP - N)]).reshape(NW, K3, C3)
    P = _sc_prop128(xs, srcp, dstp, z2d)

    g = _tc_mlp(P, ndst_col, nsrc_col, W1, b1.reshape(1, H), W2)
    q32 = _sc_prop2(g.reshape(NPF), src, dst, zflat)
    q2d = _tc_sum(q32.reshape(NW, NPF // 128, 128))
    out = _tc_final(q2d.reshape(NP, O), ndst_col, b2.reshape(1, O))
    return out[:N]


# speedup vs baseline: 4.9291x; 1.2882x over previous
import functools

import jax
import jax.numpy as jnp
from jax import lax
from jax.experimental import pallas as pl
from jax.experimental.pallas import tpu as pltpu
from jax.experimental.pallas import tpu_sc as plsc

N = 10000
E = 320000
F = 128
H = 256
O = 2

NC = 2
NS = 16
NW = NC * NS
L = 16
NP = 10240
RPT = NP // NS
EPW = E // NW
C3 = 80
K3 = 125
NPF = 2 * NP


def _mesh():
    return plsc.VectorSubcoreMesh(core_axis_name="c", subcore_axis_name="s")


def _sc_degrees(src, dst, zflat):
    @functools.partial(
        pl.kernel,
        mesh=_mesh(),
        compiler_params=pltpu.CompilerParams(needs_layout_passes=False),
        out_type=jax.ShapeDtypeStruct((NW, NPF), jnp.float32),
        scratch_types=[
            pltpu.VMEM((EPW,), jnp.int32),
            pltpu.VMEM((EPW,), jnp.int32),
            pltpu.VMEM((NPF,), jnp.float32),
        ],
    )
    def k(src_hbm, dst_hbm, zf_hbm, out_hbm, sidx_v, didx_v, cnt_v):
        c = lax.axis_index("c")
        s = lax.axis_index("s")
        wid = s * NC + c
        base = wid * EPW
        pltpu.sync_copy(zf_hbm, cnt_v)
        pltpu.sync_copy(src_hbm.at[pl.ds(base, EPW)], sidx_v)
        pltpu.sync_copy(dst_hbm.at[pl.ds(base, EPW)], didx_v)
        ones = jnp.ones((L,), jnp.float32)
        offs = jnp.full((L,), NP, jnp.int32)

        def body(g, carry):
            i0 = g * L
            plsc.addupdate_scatter(cnt_v, [sidx_v[pl.ds(i0, L)]], ones)
            plsc.addupdate_scatter(cnt_v, [didx_v[pl.ds(i0, L)] + offs], ones)
            return carry

        lax.fori_loop(0, EPW // L, body, 0)
        pltpu.sync_copy(cnt_v, out_hbm.at[wid])

    return k(src, dst, zflat)


def _sc_prop128(xs, srcp, dstp, z2d):
    @functools.partial(
        pl.kernel,
        mesh=_mesh(),
        compiler_params=pltpu.CompilerParams(needs_layout_passes=False),
        out_type=jax.ShapeDtypeStruct((NC, NP, F), jnp.float32),
        scratch_types=[
            pltpu.VMEM((K3, C3), jnp.int32),
            pltpu.VMEM((C3,), jnp.int32),
            pltpu.VMEM((C3,), jnp.int32),
            pltpu.VMEM((2, C3, F), jnp.float32),
            pltpu.VMEM_SHARED((NP, F), jnp.float32),
            pltpu.SemaphoreType.DMA,
            pltpu.SemaphoreType.DMA,
            pltpu.SemaphoreType.DMA,
            pltpu.SemaphoreType.DMA,
        ],
    )
    def k(xs_hbm, sp_hbm, dp_hbm, z_hbm, out_hbm, src_v, da_v, db_v, rows_v,
          acc_sh, g0, g1, d0, d1):
        gsem = (g0, g1)
        dsem = (d0, d1)
        dbuf = (da_v, db_v)
        c = lax.axis_index("c")
        s = lax.axis_index("s")
        wid = s * NC + c
        pltpu.sync_copy(z_hbm, acc_sh.at[pl.ds(s * RPT, RPT)])
        pltpu.sync_copy(sp_hbm.at[wid], src_v)
        plsc.subcore_barrier()

        def start_gather(j, t):
            pltpu.async_copy(xs_hbm.at[src_v.at[j]], rows_v.at[t], gsem[t])

        def wait_gather(j, t):
            pltpu.make_async_copy(
                xs_hbm.at[src_v.at[j]], rows_v.at[t], gsem[t]
            ).wait()

        dbase = wid * (K3 * C3)

        def start_didx(j, t):
            jc = dbase + jnp.minimum(j, K3 - 1) * C3
            pltpu.async_copy(dp_hbm.at[pl.ds(jc, C3)], dbuf[t], dsem[t])

        def wait_didx(j, t):
            jc = dbase + jnp.minimum(j, K3 - 1) * C3
            pltpu.make_async_copy(
                dp_hbm.at[pl.ds(jc, C3)], dbuf[t], dsem[t]
            ).wait()

        def scatter(j, t):
            pltpu.sync_copy(rows_v.at[t], acc_sh.at[dbuf[t]], add=True)

        start_didx(0, 0)
        start_didx(1, 1)
        start_gather(0, 0)

        def body(m, carry):
            j = 2 * m
            start_gather(j + 1, 1)
            wait_gather(j, 0)
            wait_didx(j, 0)
            scatter(j, 0)
            start_didx(j + 2, 0)
            start_gather(j + 2, 0)
            wait_gather(j + 1, 1)
            wait_didx(j + 1, 1)
            scatter(j + 1, 1)
            start_didx(j + 3, 1)
            return carry

        lax.fori_loop(0, (K3 - 1) // 2, body, 0)
        j = K3 - 1
        wait_gather(j, 0)
        wait_didx(j, 0)
        scatter(j, 0)
        wait_didx(K3, 1)
        plsc.subcore_barrier()
        pltpu.sync_copy(
            acc_sh.at[pl.ds(s * RPT, RPT)], out_hbm.at[c, pl.ds(s * RPT, RPT)]
        )

    return k(xs, srcp, dstp, z2d)


def _sc_prop2(gflat, src, dst, zflat):
    @functools.partial(
        pl.kernel,
        mesh=_mesh(),
        compiler_params=pltpu.CompilerParams(needs_layout_passes=False),
        out_type=jax.ShapeDtypeStruct((NW, NPF), jnp.float32),
        scratch_types=[
            pltpu.VMEM((NPF,), jnp.float32),
            pltpu.VMEM((NPF,), jnp.float32),
            pltpu.VMEM((EPW,), jnp.int32),
            pltpu.VMEM((EPW,), jnp.int32),
        ],
    )
    def k(g_hbm, src_hbm, dst_hbm, zf_hbm, out_hbm, g_v, acc_v, sidx_v, didx_v):
        c = lax.axis_index("c")
        s = lax.axis_index("s")
        wid = s * NC + c
        base = wid * EPW
        pltpu.sync_copy(zf_hbm, acc_v)
        pltpu.sync_copy(g_hbm, g_v)
        pltpu.sync_copy(src_hbm.at[pl.ds(base, EPW)], sidx_v)
        pltpu.sync_copy(dst_hbm.at[pl.ds(base, EPW)], didx_v)
        ones = jnp.full((L,), 1, jnp.int32)

        def body(g, carry):
            i0 = g * L
            si = sidx_v[pl.ds(i0, L)] * 2
            di = didx_v[pl.ds(i0, L)] * 2
            v0 = plsc.load_gather(g_v, [si])
            v1 = plsc.load_gather(g_v, [si + ones])
            plsc.addupdate_scatter(acc_v, [di], v0)
            plsc.addupdate_scatter(acc_v, [di + ones], v1)
            return carry

        lax.fori_loop(0, EPW // L, body, 0)
        pltpu.sync_copy(acc_v, out_hbm.at[wid])

    return k(gflat, src, dst, zflat)


def _tc_count_norm(cnt32):
    def body(c_ref, o_ref):
        o_ref[...] = lax.rsqrt(jnp.maximum(jnp.sum(c_ref[...], axis=0), 1.0))

    return pl.pallas_call(
        body,
        out_shape=jax.ShapeDtypeStruct((NPF // 128, 128), jnp.float32),
    )(cnt32)


def _tc_scale(x, nsrc_col):
    def body(x_ref, n_ref, o_ref):
        o_ref[pl.ds(0, N)] = x_ref[...] * n_ref[pl.ds(0, N)]

    return pl.pallas_call(
        body,
        out_shape=jax.ShapeDtypeStruct((NP, F), jnp.float32),
    )(x, nsrc_col)


def _tc_mlp(P, ndst_col, nsrc_col, W1, b1r, W2):
    R = 1024
    NB = NP // R

    def body(p_ref, nd_ref, ns_ref, w1_ref, b1_ref, w2_ref, o_ref):
        m = (p_ref[0] + p_ref[1]) * nd_ref[...]
        h = jnp.dot(m, w1_ref[...], preferred_element_type=jnp.float32)
        h = jnp.maximum(h + b1_ref[...], 0.0)
        o_ref[...] = jnp.dot(
            h * ns_ref[...], w2_ref[...], preferred_element_type=jnp.float32
        )

    return pl.pallas_call(
        body,
        grid=(NB,),
        in_specs=[
            pl.BlockSpec((NC, R, F), lambda i: (0, i, 0)),
            pl.BlockSpec((R, 1), lambda i: (i, 0)),
            pl.BlockSpec((R, 1), lambda i: (i, 0)),
            pl.BlockSpec((F, H), lambda i: (0, 0)),
            pl.BlockSpec((1, H), lambda i: (0, 0)),
            pl.BlockSpec((H, O), lambda i: (0, 0)),
        ],
        out_specs=pl.BlockSpec((R, O), lambda i: (i, 0)),
        out_shape=jax.ShapeDtypeStruct((NP, O), jnp.float32),
    )(P, ndst_col, nsrc_col, W1, b1r, W2)


def _tc_sumfinal(q32, nd2, b2row):
    def body(q_ref, nd_ref, b_ref, o_ref):
        o_ref[...] = jnp.sum(q_ref[...], axis=0) * nd_ref[...] + b_ref[...]

    return pl.pallas_call(
        body,
        out_shape=jax.ShapeDtypeStruct((NPF // 128, 128), jnp.float32),
    )(q32, nd2, b2row)


def kernel(inputs, edge_index, W1, b1, W2, b2):
    src = edge_index[0].astype(jnp.int32)
    dst = edge_index[1].astype(jnp.int32)
    zflat = jnp.zeros((NPF,), jnp.float32)
    z2d = jnp.zeros((RPT, F), jnp.float32)

    cnt32 = _sc_degrees(src, dst, zflat)
    norms2d = _tc_count_norm(cnt32.reshape(NW, NPF // 128, 128))
    norms = norms2d.reshape(NPF, 1)
    nsrc_col = norms[:NP]
    ndst_col = norms[NP:]
    xs = _tc_scale(inputs, nsrc_col)

    srcp = src.reshape(NW, K3, C3)
    P = _sc_prop128(xs, srcp, dst, z2d)

    g = _tc_mlp(P, ndst_col, nsrc_col, W1, b1.reshape(1, H), W2)
    q32 = _sc_prop2(g.reshape(NPF), src, dst, zflat)
    nd2 = jnp.repeat(norms2d.reshape(NPF)[NP:], 2).reshape(NPF // 128, 128)
    b2row = jnp.tile(b2, 128 // O).reshape(1, 128)
    out2d = _tc_sumfinal(q32.reshape(NW, NPF // 128, 128), nd2, b2row)
    return out2d.reshape(NP, O)[:N]


# speedup vs baseline: 5.3837x; 1.0922x over previous
import functools

import jax
import jax.numpy as jnp
from jax import lax
from jax.experimental import pallas as pl
from jax.experimental.pallas import tpu as pltpu
from jax.experimental.pallas import tpu_sc as plsc

N = 10000
E = 320000
F = 128
H = 256
O = 2

NC = 2
NS = 16
NW = NC * NS
L = 16
NP = 10240
RPT = NP // NS
EPW = E // NW
C3 = 80
K3 = 125
NPF = 2 * NP


def _mesh():
    return plsc.VectorSubcoreMesh(core_axis_name="c", subcore_axis_name="s")


def _sc_degrees(src, dst, zflat):
    @functools.partial(
        pl.kernel,
        mesh=_mesh(),
        compiler_params=pltpu.CompilerParams(needs_layout_passes=False),
        out_type=jax.ShapeDtypeStruct((NW, NPF), jnp.float32),
        scratch_types=[
            pltpu.VMEM((EPW,), jnp.int32),
            pltpu.VMEM((EPW,), jnp.int32),
            pltpu.VMEM((NPF,), jnp.float32),
        ],
    )
    def k(src_hbm, dst_hbm, zf_hbm, out_hbm, sidx_v, didx_v, cnt_v):
        c = lax.axis_index("c")
        s = lax.axis_index("s")
        wid = s * NC + c
        base = wid * EPW
        pltpu.sync_copy(zf_hbm, cnt_v)
        pltpu.sync_copy(src_hbm.at[pl.ds(base, EPW)], sidx_v)
        pltpu.sync_copy(dst_hbm.at[pl.ds(base, EPW)], didx_v)
        ones = jnp.ones((L,), jnp.float32)
        offs = jnp.full((L,), NP, jnp.int32)

        def body(g, carry):
            i0 = g * L
            plsc.addupdate_scatter(cnt_v, [sidx_v[pl.ds(i0, L)]], ones)
            plsc.addupdate_scatter(cnt_v, [didx_v[pl.ds(i0, L)] + offs], ones)
            return carry

        lax.fori_loop(0, EPW // L, body, 0)
        pltpu.sync_copy(cnt_v, out_hbm.at[wid])

    return k(src, dst, zflat)


def _sc_prop128(xs, srcp, dstp, z2d):
    @functools.partial(
        pl.kernel,
        mesh=_mesh(),
        compiler_params=pltpu.CompilerParams(needs_layout_passes=False),
        out_type=jax.ShapeDtypeStruct((NC, NP, F), jnp.float32),
        scratch_types=[
            pltpu.VMEM((K3, C3), jnp.int32),
            pltpu.VMEM((C3,), jnp.int32),
            pltpu.VMEM((C3,), jnp.int32),
            pltpu.VMEM((C3,), jnp.int32),
            pltpu.VMEM((3, C3, F), jnp.float32),
            pltpu.VMEM_SHARED((NP, F), jnp.float32),
            pltpu.SemaphoreType.DMA,
            pltpu.SemaphoreType.DMA,
            pltpu.SemaphoreType.DMA,
            pltpu.SemaphoreType.DMA,
            pltpu.SemaphoreType.DMA,
            pltpu.SemaphoreType.DMA,
        ],
    )
    def k(xs_hbm, sp_hbm, dp_hbm, z_hbm, out_hbm, src_v, da_v, db_v, dc_v,
          rows_v, acc_sh, g0, g1, g2, d0, d1, d2):
        gsem = (g0, g1, g2)
        dsem = (d0, d1, d2)
        dbuf = (da_v, db_v, dc_v)
        c = lax.axis_index("c")
        s = lax.axis_index("s")
        wid = s * NC + c
        pltpu.sync_copy(z_hbm, acc_sh.at[pl.ds(s * RPT, RPT)])
        pltpu.sync_copy(sp_hbm.at[wid], src_v)
        plsc.subcore_barrier()

        def start_gather(j, t):
            jc = jnp.minimum(j, K3 - 1)
            pltpu.async_copy(xs_hbm.at[src_v.at[jc]], rows_v.at[t], gsem[t])

        def wait_gather(j, t):
            jc = jnp.minimum(j, K3 - 1)
            pltpu.make_async_copy(
                xs_hbm.at[src_v.at[jc]], rows_v.at[t], gsem[t]
            ).wait()

        dbase = wid * (K3 * C3)

        def start_didx(j, t):
            jc = dbase + jnp.minimum(j, K3 - 1) * C3
            pltpu.async_copy(dp_hbm.at[pl.ds(jc, C3)], dbuf[t], dsem[t])

        def wait_didx(j, t):
            jc = dbase + jnp.minimum(j, K3 - 1) * C3
            pltpu.make_async_copy(
                dp_hbm.at[pl.ds(jc, C3)], dbuf[t], dsem[t]
            ).wait()

        def scatter(j, t):
            pltpu.sync_copy(rows_v.at[t], acc_sh.at[dbuf[t]], add=True)

        start_didx(0, 0)
        start_didx(1, 1)
        start_didx(2, 2)
        start_gather(0, 0)
        start_gather(1, 1)

        def step(j, t):
            start_gather(j + 2, (t + 2) % 3)
            wait_gather(j, t)
            wait_didx(j, t)
            scatter(j, t)
            start_didx(j + 3, t)

        step(0, 0)
        step(1, 1)

        def body(m, carry):
            j = 3 * m + 2
            step(j, 2)
            step(j + 1, 0)
            step(j + 2, 1)
            return carry

        lax.fori_loop(0, (K3 - 2) // 3, body, 0)
        wait_gather(K3, 2)
        wait_gather(K3 + 1, 0)
        wait_didx(K3, 2)
        wait_didx(K3 + 1, 0)
        wait_didx(K3 + 2, 1)
        plsc.subcore_barrier()
        pltpu.sync_copy(
            acc_sh.at[pl.ds(s * RPT, RPT)], out_hbm.at[c, pl.ds(s * RPT, RPT)]
        )

    return k(xs, srcp, dstp, z2d)


def _sc_prop2(gflat, src, dst, zflat):
    @functools.partial(
        pl.kernel,
        mesh=_mesh(),
        compiler_params=pltpu.CompilerParams(needs_layout_passes=False),
        out_type=jax.ShapeDtypeStruct((NW, NPF), jnp.float32),
        scratch_types=[
            pltpu.VMEM((NPF,), jnp.float32),
            pltpu.VMEM((NPF,), jnp.float32),
            pltpu.VMEM((EPW,), jnp.int32),
            pltpu.VMEM((EPW,), jnp.int32),
        ],
    )
    def k(g_hbm, src_hbm, dst_hbm, zf_hbm, out_hbm, g_v, acc_v, sidx_v, didx_v):
        c = lax.axis_index("c")
        s = lax.axis_index("s")
        wid = s * NC + c
        base = wid * EPW
        pltpu.sync_copy(zf_hbm, acc_v)
        pltpu.sync_copy(g_hbm, g_v)
        pltpu.sync_copy(src_hbm.at[pl.ds(base, EPW)], sidx_v)
        pltpu.sync_copy(dst_hbm.at[pl.ds(base, EPW)], didx_v)
        ones = jnp.full((L,), 1, jnp.int32)

        def body(g, carry):
            i0 = g * L
            si = sidx_v[pl.ds(i0, L)] * 2
            di = didx_v[pl.ds(i0, L)] * 2
            v0 = plsc.load_gather(g_v, [si])
            v1 = plsc.load_gather(g_v, [si + ones])
            plsc.addupdate_scatter(acc_v, [di], v0)
            plsc.addupdate_scatter(acc_v, [di + ones], v1)
            return carry

        lax.fori_loop(0, EPW // L, body, 0)
        pltpu.sync_copy(acc_v, out_hbm.at[wid])

    return k(gflat, src, dst, zflat)


def _tc_count_norm(cnt32):
    def body(c_ref, o_ref):
        o_ref[...] = lax.rsqrt(jnp.maximum(jnp.sum(c_ref[...], axis=0), 1.0))

    return pl.pallas_call(
        body,
        out_shape=jax.ShapeDtypeStruct((NPF // 128, 128), jnp.float32),
    )(cnt32)


def _tc_scale(x, nsrc_col):
    def body(x_ref, n_ref, o_ref):
        o_ref[pl.ds(0, N)] = x_ref[...] * n_ref[pl.ds(0, N)]

    return pl.pallas_call(
        body,
        out_shape=jax.ShapeDtypeStruct((NP, F), jnp.float32),
    )(x, nsrc_col)


def _tc_mlp(P, ndst_col, nsrc_col, W1, b1r, W2):
    R = 1024
    NB = NP // R

    def body(p_ref, nd_ref, ns_ref, w1_ref, b1_ref, w2_ref, o_ref):
        m = (p_ref[0] + p_ref[1]) * nd_ref[...]
        h = jnp.dot(m, w1_ref[...], preferred_element_type=jnp.float32)
        h = jnp.maximum(h + b1_ref[...], 0.0)
        o_ref[...] = jnp.dot(
            h * ns_ref[...], w2_ref[...], preferred_element_type=jnp.float32
        )

    return pl.pallas_call(
        body,
        grid=(NB,),
        in_specs=[
            pl.BlockSpec((NC, R, F), lambda i: (0, i, 0)),
            pl.BlockSpec((R, 1), lambda i: (i, 0)),
            pl.BlockSpec((R, 1), lambda i: (i, 0)),
            pl.BlockSpec((F, H), lambda i: (0, 0)),
            pl.BlockSpec((1, H), lambda i: (0, 0)),
            pl.BlockSpec((H, O), lambda i: (0, 0)),
        ],
        out_specs=pl.BlockSpec((R, O), lambda i: (i, 0)),
        out_shape=jax.ShapeDtypeStruct((NP, O), jnp.float32),
    )(P, ndst_col, nsrc_col, W1, b1r, W2)


def _tc_sumfinal(q32, nd2, b2row):
    def body(q_ref, nd_ref, b_ref, o_ref):
        o_ref[...] = jnp.sum(q_ref[...], axis=0) * nd_ref[...] + b_ref[...]

    return pl.pallas_call(
        body,
        out_shape=jax.ShapeDtypeStruct((NPF // 128, 128), jnp.float32),
    )(q32, nd2, b2row)


def kernel(inputs, edge_index, W1, b1, W2, b2):
    src = edge_index[0].astype(jnp.int32)
    dst = edge_index[1].astype(jnp.int32)
    zflat = jnp.zeros((NPF,), jnp.float32)
    z2d = jnp.zeros((RPT, F), jnp.float32)

    cnt32 = _sc_degrees(src, dst, zflat)
    norms2d = _tc_count_norm(cnt32.reshape(NW, NPF // 128, 128))
    norms = norms2d.reshape(NPF, 1)
    nsrc_col = norms[:NP]
    ndst_col = norms[NP:]
    xs = _tc_scale(inputs, nsrc_col)

    srcp = src.reshape(NW, K3, C3)
    P = _sc_prop128(xs, srcp, dst, z2d)

    g = _tc_mlp(P, ndst_col, nsrc_col, W1, b1.reshape(1, H), W2)
    q32 = _sc_prop2(g.reshape(NPF), src, dst, zflat)
    nd2 = jnp.repeat(norms2d.reshape(NPF)[NP:], 2).reshape(NPF // 128, 128)
    b2row = jnp.tile(b2, 128 // O).reshape(1, 128)
    out2d = _tc_sumfinal(q32.reshape(NW, NPF // 128, 128), nd2, b2row)
    return out2d.reshape(NP, O)[:N]


# speedup vs baseline: 5.4497x; 1.0123x over previous
import functools

import jax
import jax.numpy as jnp
from jax import lax
from jax.experimental import pallas as pl
from jax.experimental.pallas import tpu as pltpu
from jax.experimental.pallas import tpu_sc as plsc

N = 10000
E = 320000
F = 128
H = 256
O = 2

NC = 2
NS = 16
NW = NC * NS
L = 16
NP = 10240
RPT = NP // NS
EPW = E // NW
C3 = 80
K3 = 125
NPF = 2 * NP


def _mesh():
    return plsc.VectorSubcoreMesh(core_axis_name="c", subcore_axis_name="s")


def _sc_degrees(src, dst, zflat):
    @functools.partial(
        pl.kernel,
        mesh=_mesh(),
        compiler_params=pltpu.CompilerParams(needs_layout_passes=False),
        out_type=jax.ShapeDtypeStruct((NW, NPF), jnp.float32),
        scratch_types=[
            pltpu.VMEM((EPW,), jnp.int32),
            pltpu.VMEM((EPW,), jnp.int32),
            pltpu.VMEM((NPF,), jnp.float32),
            pltpu.SemaphoreType.DMA,
            pltpu.SemaphoreType.DMA,
            pltpu.SemaphoreType.DMA,
        ],
    )
    def k(src_hbm, dst_hbm, zf_hbm, out_hbm, sidx_v, didx_v, cnt_v,
          sem0, sem1, sem2):
        c = lax.axis_index("c")
        s = lax.axis_index("s")
        wid = s * NC + c
        base = wid * EPW
        cp0 = pltpu.async_copy(zf_hbm, cnt_v, sem0)
        cp1 = pltpu.async_copy(src_hbm.at[pl.ds(base, EPW)], sidx_v, sem1)
        cp2 = pltpu.async_copy(dst_hbm.at[pl.ds(base, EPW)], didx_v, sem2)
        cp0.wait()
        cp1.wait()
        cp2.wait()
        ones = jnp.ones((L,), jnp.float32)
        offs = jnp.full((L,), NP, jnp.int32)

        def body(g, carry):
            i0 = g * L
            plsc.addupdate_scatter(cnt_v, [sidx_v[pl.ds(i0, L)]], ones)
            plsc.addupdate_scatter(cnt_v, [didx_v[pl.ds(i0, L)] + offs], ones)
            return carry

        lax.fori_loop(0, EPW // L, body, 0)
        pltpu.sync_copy(cnt_v, out_hbm.at[wid])

    return k(src, dst, zflat)


def _sc_prop128(xs, srcp, dstp, z2d):
    @functools.partial(
        pl.kernel,
        mesh=_mesh(),
        compiler_params=pltpu.CompilerParams(needs_layout_passes=False),
        out_type=jax.ShapeDtypeStruct((NC, NP, F), jnp.float32),
        scratch_types=[
            pltpu.VMEM((K3, C3), jnp.int32),
            pltpu.VMEM((C3,), jnp.int32),
            pltpu.VMEM((C3,), jnp.int32),
            pltpu.VMEM((C3,), jnp.int32),
            pltpu.VMEM((3, C3, F), jnp.float32),
            pltpu.VMEM_SHARED((NP, F), jnp.float32),
            pltpu.SemaphoreType.DMA,
            pltpu.SemaphoreType.DMA,
            pltpu.SemaphoreType.DMA,
            pltpu.SemaphoreType.DMA,
            pltpu.SemaphoreType.DMA,
            pltpu.SemaphoreType.DMA,
        ],
    )
    def k(xs_hbm, sp_hbm, dp_hbm, z_hbm, out_hbm, src_v, da_v, db_v, dc_v,
          rows_v, acc_sh, g0, g1, g2, d0, d1, d2):
        gsem = (g0, g1, g2)
        dsem = (d0, d1, d2)
        dbuf = (da_v, db_v, dc_v)
        c = lax.axis_index("c")
        s = lax.axis_index("s")
        wid = s * NC + c
        pltpu.sync_copy(z_hbm, acc_sh.at[pl.ds(s * RPT, RPT)])
        pltpu.sync_copy(sp_hbm.at[wid], src_v)
        plsc.subcore_barrier()

        def start_gather(j, t):
            jc = jnp.minimum(j, K3 - 1)
            pltpu.async_copy(xs_hbm.at[src_v.at[jc]], rows_v.at[t], gsem[t])

        def wait_gather(j, t):
            jc = jnp.minimum(j, K3 - 1)
            pltpu.make_async_copy(
                xs_hbm.at[src_v.at[jc]], rows_v.at[t], gsem[t]
            ).wait()

        dbase = wid * (K3 * C3)

        def start_didx(j, t):
            jc = dbase + jnp.minimum(j, K3 - 1) * C3
            pltpu.async_copy(dp_hbm.at[pl.ds(jc, C3)], dbuf[t], dsem[t])

        def wait_didx(j, t):
            jc = dbase + jnp.minimum(j, K3 - 1) * C3
            pltpu.make_async_copy(
                dp_hbm.at[pl.ds(jc, C3)], dbuf[t], dsem[t]
            ).wait()

        def scatter(j, t):
            pltpu.sync_copy(rows_v.at[t], acc_sh.at[dbuf[t]], add=True)

        start_didx(0, 0)
        start_didx(1, 1)
        start_didx(2, 2)
        start_gather(0, 0)
        start_gather(1, 1)

        def step(j, t):
            start_gather(j + 2, (t + 2) % 3)
            wait_gather(j, t)
            wait_didx(j, t)
            scatter(j, t)
            start_didx(j + 3, t)

        step(0, 0)
        step(1, 1)

        def body(m, carry):
            j = 3 * m + 2
            step(j, 2)
            step(j + 1, 0)
            step(j + 2, 1)
            return carry

        lax.fori_loop(0, (K3 - 2) // 3, body, 0)
        wait_gather(K3, 2)
        wait_gather(K3 + 1, 0)
        wait_didx(K3, 2)
        wait_didx(K3 + 1, 0)
        wait_didx(K3 + 2, 1)
        plsc.subcore_barrier()
        pltpu.sync_copy(
            acc_sh.at[pl.ds(s * RPT, RPT)], out_hbm.at[c, pl.ds(s * RPT, RPT)]
        )

    return k(xs, srcp, dstp, z2d)


def _sc_prop2(gflat, src, dst, zflat):
    @functools.partial(
        pl.kernel,
        mesh=_mesh(),
        compiler_params=pltpu.CompilerParams(needs_layout_passes=False),
        out_type=jax.ShapeDtypeStruct((NW, NPF), jnp.float32),
        scratch_types=[
            pltpu.VMEM((NPF,), jnp.float32),
            pltpu.VMEM((NPF,), jnp.float32),
            pltpu.VMEM((EPW,), jnp.int32),
            pltpu.VMEM((EPW,), jnp.int32),
            pltpu.SemaphoreType.DMA,
            pltpu.SemaphoreType.DMA,
            pltpu.SemaphoreType.DMA,
            pltpu.SemaphoreType.DMA,
        ],
    )
    def k(g_hbm, src_hbm, dst_hbm, zf_hbm, out_hbm, g_v, acc_v, sidx_v, didx_v,
          sem0, sem1, sem2, sem3):
        c = lax.axis_index("c")
        s = lax.axis_index("s")
        wid = s * NC + c
        base = wid * EPW
        cp0 = pltpu.async_copy(zf_hbm, acc_v, sem0)
        cp1 = pltpu.async_copy(g_hbm, g_v, sem1)
        cp2 = pltpu.async_copy(src_hbm.at[pl.ds(base, EPW)], sidx_v, sem2)
        cp3 = pltpu.async_copy(dst_hbm.at[pl.ds(base, EPW)], didx_v, sem3)
        cp0.wait()
        cp1.wait()
        cp2.wait()
        cp3.wait()
        ones = jnp.full((L,), 1, jnp.int32)

        def body(g, carry):
            i0 = g * L
            si = sidx_v[pl.ds(i0, L)] * 2
            di = didx_v[pl.ds(i0, L)] * 2
            v0 = plsc.load_gather(g_v, [si])
            v1 = plsc.load_gather(g_v, [si + ones])
            plsc.addupdate_scatter(acc_v, [di], v0)
            plsc.addupdate_scatter(acc_v, [di + ones], v1)
            return carry

        lax.fori_loop(0, EPW // L, body, 0)
        pltpu.sync_copy(acc_v, out_hbm.at[wid])

    return k(gflat, src, dst, zflat)


def _tc_count_norm(cnt32):
    def body(c_ref, o_ref):
        o_ref[...] = lax.rsqrt(jnp.maximum(jnp.sum(c_ref[...], axis=0), 1.0))

    return pl.pallas_call(
        body,
        out_shape=jax.ShapeDtypeStruct((NPF // 128, 128), jnp.float32),
    )(cnt32)


def _tc_scale(x, nsrc_col):
    def body(x_ref, n_ref, o_ref):
        o_ref[pl.ds(0, N)] = x_ref[...] * n_ref[pl.ds(0, N)]

    return pl.pallas_call(
        body,
        out_shape=jax.ShapeDtypeStruct((NP, F), jnp.float32),
    )(x, nsrc_col)


def _tc_mlp(P, ndst_col, nsrc_col, W1, b1r, W2):
    R = 1024
    NB = NP // R

    def body(p_ref, nd_ref, ns_ref, w1_ref, b1_ref, w2_ref, o_ref):
        m = (p_ref[0] + p_ref[1]) * nd_ref[...]
        h = jnp.dot(m, w1_ref[...], preferred_element_type=jnp.float32)
        h = jnp.maximum(h + b1_ref[...], 0.0)
        o_ref[...] = jnp.dot(
            h * ns_ref[...], w2_ref[...], preferred_element_type=jnp.float32
        )

    return pl.pallas_call(
        body,
        grid=(NB,),
        in_specs=[
            pl.BlockSpec((NC, R, F), lambda i: (0, i, 0)),
            pl.BlockSpec((R, 1), lambda i: (i, 0)),
            pl.BlockSpec((R, 1), lambda i: (i, 0)),
            pl.BlockSpec((F, H), lambda i: (0, 0)),
            pl.BlockSpec((1, H), lambda i: (0, 0)),
            pl.BlockSpec((H, O), lambda i: (0, 0)),
        ],
        out_specs=pl.BlockSpec((R, O), lambda i: (i, 0)),
        out_shape=jax.ShapeDtypeStruct((NP, O), jnp.float32),
    )(P, ndst_col, nsrc_col, W1, b1r, W2)


def _tc_sumfinal(q32, nd2, b2row):
    def body(q_ref, nd_ref, b_ref, o_ref):
        o_ref[...] = jnp.sum(q_ref[...], axis=0) * nd_ref[...] + b_ref[...]

    return pl.pallas_call(
        body,
        out_shape=jax.ShapeDtypeStruct((NPF // 128, 128), jnp.float32),
    )(q32, nd2, b2row)


def kernel(inputs, edge_index, W1, b1, W2, b2):
    src = edge_index[0].astype(jnp.int32)
    dst = edge_index[1].astype(jnp.int32)
    zflat = jnp.zeros((NPF,), jnp.float32)
    z2d = jnp.zeros((RPT, F), jnp.float32)

    cnt32 = _sc_degrees(src, dst, zflat)
    norms2d = _tc_count_norm(cnt32.reshape(NW, NPF // 128, 128))
    norms = norms2d.reshape(NPF, 1)
    nsrc_col = norms[:NP]
    ndst_col = norms[NP:]
    xs = _tc_scale(inputs, nsrc_col)

    srcp = src.reshape(NW, K3, C3)
    P = _sc_prop128(xs, srcp, dst, z2d)

    g = _tc_mlp(P, ndst_col, nsrc_col, W1, b1.reshape(1, H), W2)
    q32 = _sc_prop2(g.reshape(NPF), src, dst, zflat)
    nd2 = jnp.repeat(ndst_col.reshape(NP), 2).reshape(NPF // 128, 128)
    b2row = jnp.tile(b2, 128 // O).reshape(1, 128)
    out2d = _tc_sumfinal(q32.reshape(NW, NPF // 128, 128), nd2, b2row)
    return out2d.reshape(NP, O)[:N]
